# Initial kernel scaffold; baseline (speedup 1.0000x reference)
import jax
import jax.numpy as jnp
from jax.experimental import pallas as pl


def kernel(prot_scalars, prot_vectors, lig_scalars, pr_pr_edge_index, lig_pr_edge_index, pr_pr_eattr, lig_pr_eattr, lig_coords, backbone_coords, W_msg_pr, b_msg_pr, attn_pr, W_msg_lig, b_msg_lig, attn_lig, W_upd, b_upd, W_vecmix, W_flp, b_flp, W_fpp, b_fpp):
    raise NotImplementedError("write your pallas kernel here")


# speedup vs baseline: 7.2562x; 7.2562x over previous
import numpy as np
import jax
import jax.numpy as jnp
from jax.experimental import pallas as pl

D = 256
DE = 128
H = 4
DH = 64
V = 4

_BE = 2000
_BN = 2000


def _dot(a, b):
    return jnp.dot(a, b, preferred_element_type=jnp.float32)


def _proj3_body(x, wa, wb, wc, oa, ob, oc):
    xv = x[...]
    oa[...] = _dot(xv, wa[...])
    ob[...] = _dot(xv, wb[...])
    oc[...] = _dot(xv, wc[...])


def _proj1_body(x, wa, oa):
    oa[...] = _dot(x[...], wa[...])


def _edge_msg_body(g1, g2, ea, w3, b, attnf, sel, exp4, ow, oe):
    msg = g1[...] + g2[...] + _dot(ea[...], w3[...]) + b[...]
    lr = jnp.where(msg > 0, msg, 0.2 * msg)
    sc = _dot(lr * attnf[...], sel[...])
    e = jnp.exp(sc)
    ef = _dot(e, exp4[...])
    ow[...] = msg * ef
    oe[...] = e


def _update_body(ps, npp, dpp, nlp, dlp, vec, wu1, wu2, wu3, bu,
                 e4, m12, sel12, expw, os_, ov, onv):
    agg_pp = npp[...] / (_dot(dpp[...], e4[...]) + 1e-9)
    agg_lp = nlp[...] / (_dot(dlp[...], e4[...]) + 1e-9)
    psv = ps[...]
    u = _dot(psv, wu1[...]) + _dot(agg_pp, wu2[...]) + _dot(agg_lp, wu3[...]) + bu[...]
    os_[...] = psv + jnp.maximum(u, 0.0)
    vo = _dot(vec[...], m12[...])
    ov[...] = vo
    n2 = _dot(vo * vo, sel12[...]) + 1e-8
    inv = 1.0 / jnp.sqrt(n2)
    onv[...] = vo * _dot(inv, expw[...])


def _edge_pp_body(gs, gd, ea, wt, wb_, bf, x1, x2, s3, o):
    fs = _dot(gs[...], x1[...])
    fd_ = _dot(gd[...], x2[...])
    fd = _dot(fs * fd_, s3[...])
    o[...] = _dot(ea[...], wt[...]) + _dot(fd, wb_[...]) + bf[...]


def _edge_lp_body(gd, ea, lc, bb, wt, wb_, bf, sel81, t8, s3b, o):
    dsp = lc[...] - bb[...]
    n2 = _dot(dsp * dsp, sel81[...]) + 1e-8
    nd = dsp / jnp.sqrt(n2)
    nde = _dot(nd, t8[...])
    ld = _dot(gd[...] * nde, s3b[...])
    o[...] = _dot(ea[...], wt[...]) + _dot(ld, wb_[...]) + bf[...]


def _full(shape):
    return pl.BlockSpec(shape, lambda i: (0, 0))


def _rows(bs, c):
    return pl.BlockSpec((bs, c), lambda i: (i, 0))


def _f32(x):
    return jax.ShapeDtypeStruct(x, jnp.float32)


def kernel(prot_scalars, prot_vectors, lig_scalars, pr_pr_edge_index,
           lig_pr_edge_index, pr_pr_eattr, lig_pr_eattr, lig_coords,
           backbone_coords, W_msg_pr, b_msg_pr, attn_pr, W_msg_lig, b_msg_lig,
           attn_lig, W_upd, b_upd, W_vecmix, W_flp, b_flp, W_fpp, b_fpp):
    n = prot_scalars.shape[0]
    nl = lig_scalars.shape[0]
    e_pp = pr_pr_eattr.shape[0]
    e_lp = lig_pr_eattr.shape[0]

    sel = np.zeros((D, 128), np.float32)
    sel[np.arange(D), np.arange(D) // DH] = 1.0
    exp4a = np.zeros((128, D), np.float32)
    exp4a[np.arange(D) // DH, np.arange(D)] = 1.0
    exp4b = exp4a[:H]
    sel12 = np.zeros((3 * V, V), np.float32)
    sel12[np.arange(3 * V), np.arange(3 * V) // 3] = 1.0
    expw = np.zeros((V, 3 * V), np.float32)
    expw[np.arange(3 * V) // 3, np.arange(3 * V)] = 1.0
    vw = np.arange(V * V)
    x1 = np.zeros((3 * V, 3 * V * V), np.float32)
    x2 = np.zeros((3 * V, 3 * V * V), np.float32)
    for v in range(V):
        for w in range(V):
            for k in range(3):
                x1[v * 3 + k, (v * V + w) * 3 + k] = 1.0
                x2[w * 3 + k, (v * V + w) * 3 + k] = 1.0
    s3 = np.zeros((3 * V * V, V * V), np.float32)
    s3[np.arange(3 * V * V), np.arange(3 * V * V) // 3] = 1.0
    sel81 = np.zeros((8, 8), np.float32)
    sel81[:3, :] = 1.0
    t8 = np.zeros((8, 3 * V), np.float32)
    for v in range(V):
        for k in range(3):
            t8[k, v * 3 + k] = 1.0
    s3b = np.zeros((3 * V, V), np.float32)
    s3b[np.arange(3 * V), np.arange(3 * V) // 3] = 1.0

    sel, exp4a, exp4b, sel12, expw, x1, x2, s3, sel81, t8, s3b = map(
        jnp.asarray, (sel, exp4a, exp4b, sel12, expw, x1, x2, s3, sel81, t8, s3b))

    psrc, pdst = pr_pr_edge_index[0], pr_pr_edge_index[1]
    lsrc, ldst = lig_pr_edge_index[0], lig_pr_edge_index[1]

    wpr1, wpr2, w3p = W_msg_pr[:D], W_msg_pr[D:2 * D], W_msg_pr[2 * D:]
    wlg1, wlg2, w3l = W_msg_lig[:D], W_msg_lig[D:2 * D], W_msg_lig[2 * D:]
    p1, p2, p3 = pl.pallas_call(
        _proj3_body,
        grid=(n // _BN,),
        in_specs=[_rows(_BN, D), _full((D, D)), _full((D, D)), _full((D, D))],
        out_specs=[_rows(_BN, D)] * 3,
        out_shape=[_f32((n, D))] * 3,
    )(prot_scalars, wpr1, wpr2, wlg2)
    l1 = pl.pallas_call(
        _proj1_body,
        grid=(nl // _BN,),
        in_specs=[_rows(_BN, D), _full((D, D))],
        out_specs=_rows(_BN, D),
        out_shape=_f32((nl, D)),
    )(lig_scalars, wlg1)

    def edge_msg(g1, g2, ea, w3, b, attnf, ne):
        return pl.pallas_call(
            _edge_msg_body,
            grid=(ne // _BE,),
            in_specs=[_rows(_BE, D), _rows(_BE, D), _rows(_BE, DE),
                      _full((DE, D)), _full((1, D)), _full((1, D)),
                      _full((D, 128)), _full((128, D))],
            out_specs=[_rows(_BE, D), _rows(_BE, 128)],
            out_shape=[_f32((ne, D)), _f32((ne, 128))],
        )(g1, g2, ea, w3, b, attnf, sel, exp4a)

    wpp, epp = edge_msg(p1[psrc], p2[pdst], pr_pr_eattr, w3p,
                        b_msg_pr.reshape(1, D), attn_pr.reshape(1, D), e_pp)
    wlp, elp = edge_msg(l1[lsrc], p3[ldst], lig_pr_eattr, w3l,
                        b_msg_lig.reshape(1, D), attn_lig.reshape(1, D), e_lp)

    num_pp = jax.ops.segment_sum(wpp, pdst, num_segments=n)
    den_pp = jax.ops.segment_sum(epp[:, :H], pdst, num_segments=n)
    num_lp = jax.ops.segment_sum(wlp, ldst, num_segments=n)
    den_lp = jax.ops.segment_sum(elp[:, :H], ldst, num_segments=n)

    wu1, wu2, wu3 = W_upd[:D], W_upd[D:2 * D], W_upd[2 * D:]
    m12 = jnp.kron(W_vecmix, jnp.eye(3, dtype=jnp.float32))
    vec_flat = prot_vectors.reshape(n, 3 * V)
    scalars_out, vout, nvf = pl.pallas_call(
        _update_body,
        grid=(n // _BN,),
        in_specs=[_rows(_BN, D), _rows(_BN, D), _rows(_BN, H), _rows(_BN, D),
                  _rows(_BN, H), _rows(_BN, 3 * V),
                  _full((D, D)), _full((D, D)), _full((D, D)), _full((1, D)),
                  _full((H, D)), _full((3 * V, 3 * V)), _full((3 * V, V)),
                  _full((V, 3 * V))],
        out_specs=[_rows(_BN, D), _rows(_BN, 3 * V), _rows(_BN, 3 * V)],
        out_shape=[_f32((n, D)), _f32((n, 3 * V)), _f32((n, 3 * V))],
    )(prot_scalars, num_pp, den_pp, num_lp, den_lp, vec_flat,
      wu1, wu2, wu3, b_upd.reshape(1, D), exp4b, m12, sel12, expw)
    vectors_out = vout.reshape(n, V, 3)

    pr_pr_out = pl.pallas_call(
        _edge_pp_body,
        grid=(e_pp // _BE,),
        in_specs=[_rows(_BE, 3 * V), _rows(_BE, 3 * V), _rows(_BE, DE),
                  _full((DE, DE)), _full((V * V, DE)), _full((1, DE)),
                  _full((3 * V, 3 * V * V)), _full((3 * V, 3 * V * V)),
                  _full((3 * V * V, V * V))],
        out_specs=_rows(_BE, DE),
        out_shape=_f32((e_pp, DE)),
    )(nvf[psrc], nvf[pdst], pr_pr_eattr, W_fpp[:DE], W_fpp[DE:],
      b_fpp.reshape(1, DE), x1, x2, s3)

    lc8 = jnp.pad(lig_coords[lsrc], ((0, 0), (0, 5)))
    bb8 = jnp.pad(backbone_coords[ldst, 1], ((0, 0), (0, 5)))
    lig_pr_out = pl.pallas_call(
        _edge_lp_body,
        grid=(e_lp // _BE,),
        in_specs=[_rows(_BE, 3 * V), _rows(_BE, DE), _rows(_BE, 8),
                  _rows(_BE, 8), _full((DE, DE)), _full((V, DE)),
                  _full((1, DE)), _full((8, 8)), _full((8, 3 * V)),
                  _full((3 * V, V))],
        out_specs=_rows(_BE, DE),
        out_shape=_f32((e_lp, DE)),
    )(nvf[ldst], lig_pr_eattr, lc8, bb8, W_flp[:DE], W_flp[DE:],
      b_flp.reshape(1, DE), sel81, t8, s3b)

    return (scalars_out, vectors_out, pr_pr_out, lig_pr_out)


# speedup vs baseline: 7.6225x; 1.0505x over previous
import numpy as np
import jax
import jax.numpy as jnp
from jax.experimental import pallas as pl

D = 256
DE = 128
H = 4
DH = 64
V = 4

_BE = 2000
_BN = 2000


def _dot(a, b):
    return jnp.dot(a, b, preferred_element_type=jnp.float32)


def _proj3_body(x, wa, wb, wc, oa, ob, oc):
    xv = x[...]
    oa[...] = _dot(xv, wa[...])
    ob[...] = _dot(xv, wb[...])
    oc[...] = _dot(xv, wc[...])


def _proj1_body(x, wa, oa):
    oa[...] = _dot(x[...], wa[...])


def _edge_msg_body(g1, g2, ea, w3, b, attnf, sel, exp4, ow, oe):
    msg = g1[...] + g2[...] + _dot(ea[...], w3[...]) + b[...]
    lr = jnp.where(msg > 0, msg, 0.2 * msg)
    sc = _dot(lr * attnf[...], sel[...])
    e = jnp.exp(sc - 4.0)
    ef = _dot(e, exp4[...])
    ow[...] = (msg * ef).astype(jnp.bfloat16)
    oe[...] = e


def _update_body(ps, npp, dpp, nlp, dlp, vec, wu1, wu2, wu3, bu,
                 e4, m12, sel12, expw, os_, ov, onv):
    agg_pp = npp[...].astype(jnp.float32) / (_dot(dpp[...], e4[...]) + 1e-9)
    agg_lp = nlp[...].astype(jnp.float32) / (_dot(dlp[...], e4[...]) + 1e-9)
    psv = ps[...]
    u = _dot(psv, wu1[...]) + _dot(agg_pp, wu2[...]) + _dot(agg_lp, wu3[...]) + bu[...]
    os_[...] = psv + jnp.maximum(u, 0.0)
    vo = _dot(vec[...], m12[...])
    ov[...] = vo
    n2 = _dot(vo * vo, sel12[...]) + 1e-8
    inv = 1.0 / jnp.sqrt(n2)
    onv[...] = vo * _dot(inv, expw[...])


def _edge_pp_body(gs, gd, ea, wt, wb_, bf, x1, x2, s3, o):
    fs = _dot(gs[...], x1[...])
    fd_ = _dot(gd[...], x2[...])
    fd = _dot(fs * fd_, s3[...])
    o[...] = _dot(ea[...], wt[...]) + _dot(fd, wb_[...]) + bf[...]


def _edge_lp_body(gd, ea, lc, bb, wt, wb_, bf, sel81, t8, s3b, o):
    dsp = lc[...] - bb[...]
    n2 = _dot(dsp * dsp, sel81[...]) + 1e-8
    nd = dsp / jnp.sqrt(n2)
    nde = _dot(nd, t8[...])
    ld = _dot(gd[...] * nde, s3b[...])
    o[...] = _dot(ea[...], wt[...]) + _dot(ld, wb_[...]) + bf[...]


def _full(shape):
    return pl.BlockSpec(shape, lambda i: (0, 0))


def _rows(bs, c):
    return pl.BlockSpec((bs, c), lambda i: (i, 0))


def _f32(x):
    return jax.ShapeDtypeStruct(x, jnp.float32)


def kernel(prot_scalars, prot_vectors, lig_scalars, pr_pr_edge_index,
           lig_pr_edge_index, pr_pr_eattr, lig_pr_eattr, lig_coords,
           backbone_coords, W_msg_pr, b_msg_pr, attn_pr, W_msg_lig, b_msg_lig,
           attn_lig, W_upd, b_upd, W_vecmix, W_flp, b_flp, W_fpp, b_fpp):
    n = prot_scalars.shape[0]
    nl = lig_scalars.shape[0]
    e_pp = pr_pr_eattr.shape[0]
    e_lp = lig_pr_eattr.shape[0]

    sel = np.zeros((D, 128), np.float32)
    sel[np.arange(D), np.arange(D) // DH] = 1.0
    exp4a = np.zeros((128, D), np.float32)
    exp4a[np.arange(D) // DH, np.arange(D)] = 1.0
    exp4b = exp4a[:H]
    sel12 = np.zeros((3 * V, V), np.float32)
    sel12[np.arange(3 * V), np.arange(3 * V) // 3] = 1.0
    expw = np.zeros((V, 3 * V), np.float32)
    expw[np.arange(3 * V) // 3, np.arange(3 * V)] = 1.0
    vw = np.arange(V * V)
    x1 = np.zeros((3 * V, 3 * V * V), np.float32)
    x2 = np.zeros((3 * V, 3 * V * V), np.float32)
    for v in range(V):
        for w in range(V):
            for k in range(3):
                x1[v * 3 + k, (v * V + w) * 3 + k] = 1.0
                x2[w * 3 + k, (v * V + w) * 3 + k] = 1.0
    s3 = np.zeros((3 * V * V, V * V), np.float32)
    s3[np.arange(3 * V * V), np.arange(3 * V * V) // 3] = 1.0
    sel81 = np.zeros((8, 8), np.float32)
    sel81[:3, :] = 1.0
    t8 = np.zeros((8, 3 * V), np.float32)
    for v in range(V):
        for k in range(3):
            t8[k, v * 3 + k] = 1.0
    s3b = np.zeros((3 * V, V), np.float32)
    s3b[np.arange(3 * V), np.arange(3 * V) // 3] = 1.0

    sel, exp4a, exp4b, sel12, expw, x1, x2, s3, sel81, t8, s3b = map(
        jnp.asarray, (sel, exp4a, exp4b, sel12, expw, x1, x2, s3, sel81, t8, s3b))

    psrc, pdst = pr_pr_edge_index[0], pr_pr_edge_index[1]
    lsrc, ldst = lig_pr_edge_index[0], lig_pr_edge_index[1]

    wpr1, wpr2, w3p = W_msg_pr[:D], W_msg_pr[D:2 * D], W_msg_pr[2 * D:]
    wlg1, wlg2, w3l = W_msg_lig[:D], W_msg_lig[D:2 * D], W_msg_lig[2 * D:]
    p1, p2, p3 = pl.pallas_call(
        _proj3_body,
        grid=(n // _BN,),
        in_specs=[_rows(_BN, D), _full((D, D)), _full((D, D)), _full((D, D))],
        out_specs=[_rows(_BN, D)] * 3,
        out_shape=[_f32((n, D))] * 3,
    )(prot_scalars, wpr1, wpr2, wlg2)
    l1 = pl.pallas_call(
        _proj1_body,
        grid=(nl // _BN,),
        in_specs=[_rows(_BN, D), _full((D, D))],
        out_specs=_rows(_BN, D),
        out_shape=_f32((nl, D)),
    )(lig_scalars, wlg1)

    def edge_msg(g1, g2, ea, w3, b, attnf, ne):
        return pl.pallas_call(
            _edge_msg_body,
            grid=(ne // _BE,),
            in_specs=[_rows(_BE, D), _rows(_BE, D), _rows(_BE, DE),
                      _full((DE, D)), _full((1, D)), _full((1, D)),
                      _full((D, 128)), _full((128, D))],
            out_specs=[_rows(_BE, D), _rows(_BE, 128)],
            out_shape=[jax.ShapeDtypeStruct((ne, D), jnp.bfloat16),
                       _f32((ne, 128))],
        )(g1, g2, ea, w3, b, attnf, sel, exp4a)

    wpp, epp = edge_msg(p1[psrc], p2[pdst], pr_pr_eattr, w3p,
                        b_msg_pr.reshape(1, D), attn_pr.reshape(1, D), e_pp)
    wlp, elp = edge_msg(l1[lsrc], p3[ldst], lig_pr_eattr, w3l,
                        b_msg_lig.reshape(1, D), attn_lig.reshape(1, D), e_lp)

    num_pp = jax.ops.segment_sum(wpp, pdst, num_segments=n)
    den_pp = jax.ops.segment_sum(epp[:, :H], pdst, num_segments=n)
    num_lp = jax.ops.segment_sum(wlp, ldst, num_segments=n)
    den_lp = jax.ops.segment_sum(elp[:, :H], ldst, num_segments=n)

    wu1, wu2, wu3 = W_upd[:D], W_upd[D:2 * D], W_upd[2 * D:]
    m12 = jnp.kron(W_vecmix, jnp.eye(3, dtype=jnp.float32))
    vec_flat = prot_vectors.reshape(n, 3 * V)
    scalars_out, vout, nvf = pl.pallas_call(
        _update_body,
        grid=(n // _BN,),
        in_specs=[_rows(_BN, D), _rows(_BN, D), _rows(_BN, H), _rows(_BN, D),
                  _rows(_BN, H), _rows(_BN, 3 * V),
                  _full((D, D)), _full((D, D)), _full((D, D)), _full((1, D)),
                  _full((H, D)), _full((3 * V, 3 * V)), _full((3 * V, V)),
                  _full((V, 3 * V))],
        out_specs=[_rows(_BN, D), _rows(_BN, 3 * V), _rows(_BN, 3 * V)],
        out_shape=[_f32((n, D)), _f32((n, 3 * V)), _f32((n, 3 * V))],
    )(prot_scalars, num_pp, den_pp, num_lp, den_lp, vec_flat,
      wu1, wu2, wu3, b_upd.reshape(1, D), exp4b, m12, sel12, expw)
    vectors_out = vout.reshape(n, V, 3)

    pr_pr_out = pl.pallas_call(
        _edge_pp_body,
        grid=(e_pp // _BE,),
        in_specs=[_rows(_BE, 3 * V), _rows(_BE, 3 * V), _rows(_BE, DE),
                  _full((DE, DE)), _full((V * V, DE)), _full((1, DE)),
                  _full((3 * V, 3 * V * V)), _full((3 * V, 3 * V * V)),
                  _full((3 * V * V, V * V))],
        out_specs=_rows(_BE, DE),
        out_shape=_f32((e_pp, DE)),
    )(nvf[psrc], nvf[pdst], pr_pr_eattr, W_fpp[:DE], W_fpp[DE:],
      b_fpp.reshape(1, DE), x1, x2, s3)

    lc8 = jnp.pad(lig_coords[lsrc], ((0, 0), (0, 5)))
    bb8 = jnp.pad(backbone_coords[ldst, 1], ((0, 0), (0, 5)))
    lig_pr_out = pl.pallas_call(
        _edge_lp_body,
        grid=(e_lp // _BE,),
        in_specs=[_rows(_BE, 3 * V), _rows(_BE, DE), _rows(_BE, 8),
                  _rows(_BE, 8), _full((DE, DE)), _full((V, DE)),
                  _full((1, DE)), _full((8, 8)), _full((8, 3 * V)),
                  _full((3 * V, V))],
        out_specs=_rows(_BE, DE),
        out_shape=_f32((e_lp, DE)),
    )(nvf[ldst], lig_pr_eattr, lc8, bb8, W_flp[:DE], W_flp[DE:],
      b_flp.reshape(1, DE), sel81, t8, s3b)

    return (scalars_out, vectors_out, pr_pr_out, lig_pr_out)
